# Initial kernel scaffold; baseline (speedup 1.0000x reference)
import jax
import jax.numpy as jnp
from jax.experimental import pallas as pl


def kernel(x, params):
    raise NotImplementedError("write your pallas kernel here")


# speedup vs baseline: 1.0007x; 1.0007x over previous
import functools

import jax
import jax.numpy as jnp
import numpy as np
from jax.experimental import pallas as pl
from jax.experimental.pallas import tpu as pltpu

B, N, K1, K2, KADJ, DEPTH = 2, 2048, 40, 20, 32, 2
IN_CH, OUT_CH, DIM, FEAT = 12, 8, 64, 128
EPS = 1e-5


def _lrelu(x):
    return jnp.where(x >= 0, x, 0.2 * x)


def _bn(x, g, b):
    shp = (1, -1) + (1,) * (x.ndim - 2)
    return x / np.sqrt(1.0 + EPS) * g.reshape(shp) + b.reshape(shp)


def _conv1(x, w):
    return jnp.einsum('oc,bcn->bon', w, x)


def _conv2(x, w):
    return jnp.einsum('oc,bcnk->bonk', w, x)


def _knn_idx(x, k):
    inner = -2.0 * jnp.einsum('bcn,bcm->bnm', x, x)
    xx = jnp.sum(x * x, axis=1)
    pd = -xx[:, :, None] - inner - xx[:, None, :]
    return jax.lax.top_k(pd, k)[1]


def _graph_feature(x, idx):
    xt = jnp.transpose(x, (0, 2, 1))
    feat = jax.vmap(lambda f, i: f[i])(xt, idx)
    center = jnp.broadcast_to(xt[:, :, None, :], feat.shape)
    out = jnp.concatenate([feat - center, center], axis=-1)
    return jnp.transpose(out, (0, 3, 1, 2))


def _build_adj(coords, k):
    idx = _knn_idx(coords, k)
    b_ = coords.shape[0]
    n_ = coords.shape[2]
    bi = jnp.arange(b_)[:, None, None]
    ni = jnp.arange(n_)[None, :, None]
    adj = jnp.zeros((b_, n_, n_), jnp.float32).at[bi, ni, idx].set(1.0)
    return jnp.maximum(adj, jnp.transpose(adj, (0, 2, 1)))


def _stream_head(x, p):
    h = _lrelu(_bn(_conv1(x, p['w1']), p['g1'], p['b1']))
    return _conv1(h, p['w2'])


def _mgm(x, p):
    idx40 = _knn_idx(x, K1)
    f1 = _graph_feature(x, idx40)
    f1 = _lrelu(_bn(_conv2(f1, p['w1']), p['g1'], p['b1']))
    f1 = _lrelu(_bn(_conv2(f1, p['w2']), p['g2'], p['b2']))
    xk1 = jnp.max(f1, axis=-1)
    f2 = _graph_feature(x, idx40[..., :K2])
    f2 = _lrelu(_bn(_conv2(f2, p['w3']), p['g3'], p['b3']))
    f2 = _lrelu(_bn(_conv2(f2, p['w4']), p['g4'], p['b4']))
    xk1t = jnp.repeat(xk1[:, :, :, None], K2, axis=-1)
    out = jnp.concatenate([f2, xk1t], axis=1)
    out = _lrelu(_bn(_conv2(out, p['w5']), p['g5'], p['b5']))
    return jnp.max(out, axis=-1)


def _gcn(x, adj, p):
    h = _lrelu(_bn(_conv1(x, p['wh']), p['gh'], p['bh']))
    sc = h
    for gp in p['layers']:
        t = _conv1(h, gp['w1'])
        t = _lrelu(_bn(t, gp['g'], gp['b']))
        t = _conv1(t, gp['w2'])
        h = jnp.einsum('bcn,bnm->bcm', t, adj)
    return _lrelu(_bn(_conv1(h, p['wt']), p['gt'], p['bt'])) + sc


def _cross_fusion(xc, xn, p):
    ac = jax.nn.sigmoid(_conv1(xc, p['wsc']) + p['bsc'].reshape(1, -1, 1))
    an = jax.nn.sigmoid(_conv1(xn, p['wsn']) + p['bsn'].reshape(1, -1, 1))
    xpn = xn + xn * ac
    xpc = xc + xc * an
    fs = xpc + xpn
    y = jnp.mean(fs, axis=2, keepdims=True)
    y = jax.nn.relu(_conv1(y, p['wse1']))
    y = jax.nn.sigmoid(_conv1(y, p['wse2']))
    return fs * y


def _pred_inner(f_ref, w_ref, b_ref, o_ref):
    o_ref[0] = (
        jnp.dot(w_ref[...], f_ref[0], preferred_element_type=jnp.float32)
        + b_ref[...].reshape(-1, 1)
    )


def kernel(x, params):
    xc, xn = x[:, :IN_CH, :], x[:, IN_CH:, :]
    c = _stream_head(xc, params['head_c'])
    n = _stream_head(xn, params['head_n'])
    c = _mgm(c, params['mgm_c'])
    n = _mgm(n, params['mgm_n'])
    adj = _build_adj(xc[:, :3, :], KADJ)
    c = _gcn(c, adj, params['gcn_c'])
    n = _gcn(n, adj, params['gcn_n'])
    f = _cross_fusion(c, n, params['fuse'])
    out = pl.pallas_call(
        _pred_inner,
        grid=(B,),
        in_specs=[
            pl.BlockSpec((1, FEAT, N), lambda i: (i, 0, 0)),
            pl.BlockSpec((OUT_CH, FEAT), lambda i: (0, 0)),
            pl.BlockSpec((OUT_CH,), lambda i: (0,)),
        ],
        out_specs=pl.BlockSpec((1, OUT_CH, N), lambda i: (i, 0, 0)),
        out_shape=jax.ShapeDtypeStruct((B, OUT_CH, N), jnp.float32),
    )(f, params['wpred'], params['bpred'])
    return out


# speedup vs baseline: 3.1517x; 3.1494x over previous
import functools

import jax
import jax.numpy as jnp
import numpy as np
from jax import lax
from jax.experimental import pallas as pl
from jax.experimental.pallas import tpu as pltpu
from jax.experimental.pallas import tpu_sc as plsc

B, N, K1, K2, KADJ, DEPTH = 2, 2048, 40, 20, 32, 2
IN_CH, OUT_CH, DIM, FEAT = 12, 8, 64, 128
EPS = 1e-5
SQ = np.sqrt(1.0 + EPS)
NBLK = 8
BN = N // NBLK


def _lrelu(x):
    return jnp.where(x >= 0, x, 0.2 * x)


def _dot_t(a, w):
    return lax.dot_general(a, w, (((a.ndim - 1,), (1,)), ((), ())),
                           preferred_element_type=jnp.float32)


def _sigmoid(x):
    return 1.0 / (1.0 + jnp.exp(-x))


def _f2key(x):
    b = lax.bitcast_convert_type(x, jnp.int32)
    return b ^ (lax.shift_right_arithmetic(b, 31) & jnp.int32(0x7FFFFFFF))


def _head(xs, w1, g1, b1, w2):
    h = lax.dot_general(xs, w1, (((0,), (1,)), ((), ())),
                        preferred_element_type=jnp.float32)
    h = _lrelu(h / SQ * g1 + b1)
    return _dot_t(h, w2)


def _pd_block(cb, call, i):
    g = lax.dot_general(cb, call, (((1,), (1,)), ((), ())),
                        preferred_element_type=jnp.float32)
    inner = -2.0 * g
    xxb = jnp.sum(cb * cb, axis=1, keepdims=True)
    xxa = jnp.sum(call * call, axis=1)[None, :]
    return (-xxb - inner) - xxa


def _prep_kernel(x_ref, w1c, g1c, b1c, w2c, w1n, g1n, b1n, w2n,
                 c_out, n_out, pdc_out, pdn_out, ka_out, tk_out,
                 c_s, n_s, x3_s):
    i = pl.program_id(1)
    xs = x_ref[0]
    xc = xs[:IN_CH]
    xn = xs[IN_CH:]
    c = _head(xc, w1c[...], g1c[...], b1c[...], w2c[...])
    n = _head(xn, w1n[...], g1n[...], b1n[...], w2n[...])
    c_s[...] = c
    n_s[...] = n
    x3_s[...] = jnp.transpose(xc[:3])
    rows = pl.ds(i * BN, BN)
    cb = c_s[rows, :]
    nb = n_s[rows, :]
    c_out[0] = cb
    n_out[0] = nb
    pdc_out[0] = _pd_block(cb, c, i)
    pdn_out[0] = _pd_block(nb, n, i)
    xct = jnp.transpose(xc[:3])
    ka = _f2key(_pd_block(x3_s[rows, :], xct, i))
    ka_out[0] = ka
    lo0 = jnp.min(ka, axis=1, keepdims=True)
    hi0 = jnp.max(ka, axis=1, keepdims=True)

    def body(_, carry):
        lo, hi = carry
        mid = lo + lax.shift_right_logical(hi - lo + 1, 1)
        cnt = jnp.sum((ka >= mid).astype(jnp.int32), axis=1, keepdims=True)
        ge = cnt >= KADJ
        return jnp.where(ge, mid, lo), jnp.where(ge, hi, mid - 1)

    lo, _ = lax.fori_loop(0, 32, body, (lo0, hi0))
    tk_out[0] = jnp.reshape(lo, (1, BN))


def _edge_kernel(g_ref, ctr_ref, w1a, w1b, g1, b1, w2, g2, b2,
                 w3a, w3b, g3, b3, w4, g4, b4, w5a, w5b, g5, b5, xs_out):
    G = g_ref[0][:, :DIM]
    C = ctr_ref[0]
    CB = jnp.broadcast_to(C[:, None, :], (BN, K1, DIM)).reshape(BN * K1, DIM)
    D = G - CB
    h = _lrelu((_dot_t(D, w1a[...]) + _dot_t(CB, w1b[...])) / SQ * g1[...] + b1[...])
    h = _lrelu(_dot_t(h, w2[...]) / SQ * g2[...] + b2[...])
    xk1 = jnp.max(h.reshape(BN, K1, DIM), axis=1)
    G2 = G.reshape(BN, K1, DIM)[:, :K2, :].reshape(BN * K2, DIM)
    CB2 = jnp.broadcast_to(C[:, None, :], (BN, K2, DIM)).reshape(BN * K2, DIM)
    D2 = G2 - CB2
    h3 = _lrelu((_dot_t(D2, w3a[...]) + _dot_t(CB2, w3b[...])) / SQ * g3[...] + b3[...])
    h4 = _lrelu(_dot_t(h3, w4[...]) / SQ * g4[...] + b4[...])
    xk1r = jnp.broadcast_to(xk1[:, None, :], (BN, K2, DIM)).reshape(BN * K2, DIM)
    h5 = _lrelu((_dot_t(h4, w5a[...]) + _dot_t(xk1r, w5b[...])) / SQ * g5[...] + b5[...])
    xs_out[0] = jnp.max(h5.reshape(BN, K2, DIM), axis=1)


def _adj_kernel(k_ref, kt_ref, trow_ref, tcol_ref, a_out):
    kb = k_ref[0]
    ktb = jnp.transpose(kt_ref[0], (1, 0))
    m = (kb >= tcol_ref[0]) | (ktb >= trow_ref[0])
    a_out[0] = m.astype(jnp.float32)


def _gcn(xs, A, wh, gh, bh, lw, wt, gt, bt):
    h = _lrelu(_dot_t(xs, wh) / SQ * gh + bh)
    sc = h
    for (w1, g, b, w2) in lw:
        tt = _lrelu(_dot_t(h, w1) / SQ * g + b)
        tt = _dot_t(tt, w2)
        h = lax.dot_general(A, tt, (((0,), (0,)), ((), ())),
                            preferred_element_type=jnp.float32)
    return _lrelu(_dot_t(h, wt) / SQ * gt + bt) + sc


def _final_kernel(xsc_ref, xsn_ref, a_ref,
                  whc, ghc, bhc, c10, cg0, cb0, c20, c11, cg1, cb1, c21,
                  wtc, gtc, btc,
                  whn, ghn, bhn, n10, ng0, nb0, n20, n11, ng1, nb1, n21,
                  wtn, gtn, btn,
                  wsc, bsc, wsn, bsn, wse1, wse2, wpred, bpred, out_ref):
    A = a_ref[0]
    xc = _gcn(xsc_ref[0], A, whc[...], ghc[...], bhc[...],
              [(c10[...], cg0[...], cb0[...], c20[...]),
               (c11[...], cg1[...], cb1[...], c21[...])],
              wtc[...], gtc[...], btc[...])
    xn = _gcn(xsn_ref[0], A, whn[...], ghn[...], bhn[...],
              [(n10[...], ng0[...], nb0[...], n20[...]),
               (n11[...], ng1[...], nb1[...], n21[...])],
              wtn[...], gtn[...], btn[...])
    zc = jnp.sum(xc * wsc[...], axis=1, keepdims=True)
    zn = jnp.sum(xn * wsn[...], axis=1, keepdims=True)
    ac = _sigmoid(jnp.broadcast_to(zc, (N, FEAT)) + bsc[...])
    an = _sigmoid(jnp.broadcast_to(zn, (N, FEAT)) + bsn[...])
    xpn = xn + xn * ac
    xpc = xc + xc * an
    fs = xpc + xpn
    y = jnp.mean(fs, axis=0, keepdims=True)
    y = jnp.maximum(_dot_t(y, wse1[...]), 0.0)
    y = _sigmoid(_dot_t(y, wse2[...]))
    f = fs * y
    out_ref[0] = _dot_t(f, wpred[...]) + bpred[...]


NW = 32
GCH = 128
NIDX = B * N * K1


def _sc_gather_body(table_ref, idx_ref, out_ref, idx_v, rows_v, sem):
    wid = lax.axis_index("s") * 2 + lax.axis_index("c")
    per_w = NIDX // NW

    def body(ci, carry):
        base = wid * per_w + ci * GCH
        pltpu.sync_copy(idx_ref.at[pl.ds(base, GCH)], idx_v)
        pltpu.async_copy(table_ref.at[idx_v], rows_v, sem).wait()
        pltpu.sync_copy(rows_v, out_ref.at[pl.ds(base, GCH)])
        return carry

    lax.fori_loop(0, per_w // GCH, body, 0)


def _sc_gather(table, idx):
    k = pl.kernel(
        _sc_gather_body,
        mesh=plsc.VectorSubcoreMesh(core_axis_name="c", subcore_axis_name="s"),
        out_type=jax.ShapeDtypeStruct((NIDX, FEAT), jnp.float32),
        scratch_types=[
            pltpu.VMEM((GCH,), jnp.int32),
            pltpu.VMEM((GCH, FEAT), jnp.float32),
            pltpu.SemaphoreType.DMA,
        ],
    )
    return k(table, idx)


def _full(bs):
    return pl.BlockSpec(bs, lambda b, i: (0,) * len(bs))


def _v(p):
    return p.reshape(1, -1)


def kernel(x, params):
    hc, hn = params['head_c'], params['head_n']
    c_nm, n_nm, pd_c, pd_n, ka, tk = pl.pallas_call(
        _prep_kernel,
        grid=(B, NBLK),
        in_specs=[pl.BlockSpec((1, 2 * IN_CH, N), lambda b, i: (b, 0, 0))]
        + [_full(s) for s in [(DIM, IN_CH), (1, DIM), (1, DIM), (DIM, DIM)] * 2],
        out_specs=[
            pl.BlockSpec((1, BN, DIM), lambda b, i: (b, i, 0)),
            pl.BlockSpec((1, BN, DIM), lambda b, i: (b, i, 0)),
            pl.BlockSpec((1, BN, N), lambda b, i: (b, i, 0)),
            pl.BlockSpec((1, BN, N), lambda b, i: (b, i, 0)),
            pl.BlockSpec((1, BN, N), lambda b, i: (b, i, 0)),
            pl.BlockSpec((1, 1, BN), lambda b, i: (b * NBLK + i, 0, 0)),
        ],
        out_shape=[
            jax.ShapeDtypeStruct((B, N, DIM), jnp.float32),
            jax.ShapeDtypeStruct((B, N, DIM), jnp.float32),
            jax.ShapeDtypeStruct((B, N, N), jnp.float32),
            jax.ShapeDtypeStruct((B, N, N), jnp.float32),
            jax.ShapeDtypeStruct((B, N, N), jnp.int32),
            jax.ShapeDtypeStruct((B * NBLK, 1, BN), jnp.int32),
        ],
        scratch_shapes=[
            pltpu.VMEM((N, DIM), jnp.float32),
            pltpu.VMEM((N, DIM), jnp.float32),
            pltpu.VMEM((N, 3), jnp.float32),
        ],
    )(x, hc['w1'], _v(hc['g1']), _v(hc['b1']), hc['w2'],
      hn['w1'], _v(hn['g1']), _v(hn['b1']), hn['w2'])

    idx_c = lax.top_k(pd_c, K1)[1]
    idx_n = lax.top_k(pd_n, K1)[1]
    boff = (jnp.arange(B, dtype=jnp.int32) * N)[:, None, None]
    pad = jnp.zeros((B * N, FEAT - DIM), jnp.float32)
    tab_c = jnp.concatenate([c_nm.reshape(B * N, DIM), pad], axis=1)
    tab_n = jnp.concatenate([n_nm.reshape(B * N, DIM), pad], axis=1)
    gath_c = _sc_gather(tab_c, (idx_c + boff).reshape(NIDX)
                        ).reshape(B, N * K1, FEAT)
    gath_n = _sc_gather(tab_n, (idx_n + boff).reshape(NIDX)
                        ).reshape(B, N * K1, FEAT)

    def edge(gath, ctr, p):
        ws = []
        for nm, cin in zip(['w1', 'w2', 'w3', 'w4', 'w5'],
                           [2 * DIM, DIM, 2 * DIM, DIM, 2 * DIM]):
            w = p[nm]
            if cin == 2 * DIM:
                ws += [w[:, :DIM], w[:, DIM:]]
            else:
                ws += [w]
        args = [ws[0], ws[1], _v(p['g1']), _v(p['b1']), ws[2], _v(p['g2']), _v(p['b2']),
                ws[3], ws[4], _v(p['g3']), _v(p['b3']), ws[5], _v(p['g4']), _v(p['b4']),
                ws[6], ws[7], _v(p['g5']), _v(p['b5'])]
        return pl.pallas_call(
            _edge_kernel,
            grid=(B, NBLK),
            in_specs=[pl.BlockSpec((1, BN * K1, FEAT), lambda b, i: (b, i, 0)),
                      pl.BlockSpec((1, BN, DIM), lambda b, i: (b, i, 0))]
            + [_full(a.shape) for a in args],
            out_specs=pl.BlockSpec((1, BN, DIM), lambda b, i: (b, i, 0)),
            out_shape=jax.ShapeDtypeStruct((B, N, DIM), jnp.float32),
        )(gath, ctr, *args)

    xs_c = edge(gath_c, c_nm, params['mgm_c'])
    xs_n = edge(gath_n, n_nm, params['mgm_n'])

    t = tk.reshape(B, N)
    trow = t.reshape(B, 1, N)
    tcol = t.reshape(B, N, 1)
    A = pl.pallas_call(
        _adj_kernel,
        grid=(B, NBLK),
        in_specs=[pl.BlockSpec((1, BN, N), lambda b, i: (b, i, 0)),
                  pl.BlockSpec((1, N, BN), lambda b, i: (b, 0, i)),
                  pl.BlockSpec((1, 1, N), lambda b, i: (b, 0, 0)),
                  pl.BlockSpec((1, BN, 1), lambda b, i: (b, i, 0))],
        out_specs=pl.BlockSpec((1, BN, N), lambda b, i: (b, i, 0)),
        out_shape=jax.ShapeDtypeStruct((B, N, N), jnp.float32),
    )(ka, ka, trow, tcol)

    gc, gn, fu = params['gcn_c'], params['gcn_n'], params['fuse']
    gargs = []
    for g in (gc, gn):
        gargs += [g['wh'], _v(g['gh']), _v(g['bh'])]
        for l in g['layers']:
            gargs += [l['w1'], _v(l['g']), _v(l['b']), l['w2']]
        gargs += [g['wt'], _v(g['gt']), _v(g['bt'])]
    fargs = [fu['wsc'], jnp.broadcast_to(_v(fu['bsc']), (1, FEAT)),
             fu['wsn'], jnp.broadcast_to(_v(fu['bsn']), (1, FEAT)),
             fu['wse1'], fu['wse2'], params['wpred'], _v(params['bpred'])]
    allargs = gargs + fargs
    outs = []
    for b in range(B):
        outs.append(pl.pallas_call(
            _final_kernel,
            grid=(1,),
            in_specs=[pl.BlockSpec((1, N, DIM), lambda i: (0, 0, 0)),
                      pl.BlockSpec((1, N, DIM), lambda i: (0, 0, 0)),
                      pl.BlockSpec((1, N, N), lambda i: (0, 0, 0))]
            + [pl.BlockSpec(a.shape, lambda i: (0,) * a.ndim) for a in allargs],
            out_specs=pl.BlockSpec((1, N, OUT_CH), lambda i: (0, 0, 0)),
            out_shape=jax.ShapeDtypeStruct((1, N, OUT_CH), jnp.float32),
        )(xs_c[b:b + 1], xs_n[b:b + 1], A[b:b + 1], *allargs))
    out_nm = jnp.concatenate(outs, axis=0)
    return jnp.transpose(out_nm, (0, 2, 1))


# speedup vs baseline: 5.3883x; 1.7096x over previous
import functools

import jax
import jax.numpy as jnp
import numpy as np
from jax import lax
from jax.experimental import pallas as pl
from jax.experimental.pallas import tpu as pltpu
from jax.experimental.pallas import tpu_sc as plsc

B, N, K1, K2, KADJ, DEPTH = 2, 2048, 40, 20, 32, 2
IN_CH, OUT_CH, DIM, FEAT = 12, 8, 64, 128
EPS = 1e-5
SQ = np.sqrt(1.0 + EPS)
NBLK = 8
BN = N // NBLK


def _lrelu(x):
    return jnp.where(x >= 0, x, 0.2 * x)


def _dot_t(a, w):
    return lax.dot_general(a, w, (((a.ndim - 1,), (1,)), ((), ())),
                           preferred_element_type=jnp.float32)


def _sigmoid(x):
    return 1.0 / (1.0 + jnp.exp(-x))


def _f2key(x):
    b = lax.bitcast_convert_type(x, jnp.int32)
    return b ^ (lax.shift_right_arithmetic(b, 31) & jnp.int32(0x7FFFFFFF))


def _head(xs, w1, g1, b1, w2):
    h = lax.dot_general(xs, w1, (((0,), (1,)), ((), ())),
                        preferred_element_type=jnp.float32)
    h = _lrelu(h / SQ * g1 + b1)
    return _dot_t(h, w2)


def _pd_block(cb, call, i):
    g = lax.dot_general(cb, call, (((1,), (1,)), ((), ())),
                        preferred_element_type=jnp.float32)
    inner = -2.0 * g
    xxb = jnp.sum(cb * cb, axis=1, keepdims=True)
    xxa = jnp.sum(call * call, axis=1)[None, :]
    return (-xxb - inner) - xxa


def _topk_idx(keys):
    jio = lax.broadcasted_iota(jnp.int32, keys.shape, 1)
    kio = lax.broadcasted_iota(jnp.int32, (keys.shape[0], K1), 1)
    imin = jnp.int32(-2147483648)

    def body(r, carry):
        ks, acc = carry
        m = jnp.max(ks, axis=1, keepdims=True)
        am = jnp.min(jnp.where(ks == m, jio, jnp.int32(N)), axis=1,
                     keepdims=True)
        acc = jnp.where(kio == r, am, acc)
        ks = jnp.where(jio == am, imin, ks)
        return ks, acc

    _, acc = lax.fori_loop(
        0, K1, body, (keys, jnp.zeros((keys.shape[0], K1), jnp.int32)))
    return acc


def _prep_kernel(x_ref, w1c, g1c, b1c, w2c, w1n, g1n, b1n, w2n,
                 c_out, n_out, idxc_out, idxn_out, ka_out, tk_out,
                 c_s, n_s, x3_s):
    i = pl.program_id(1)
    xs = x_ref[0]
    xc = xs[:IN_CH]
    xn = xs[IN_CH:]
    c = _head(xc, w1c[...], g1c[...], b1c[...], w2c[...])
    n = _head(xn, w1n[...], g1n[...], b1n[...], w2n[...])
    c_s[...] = c
    n_s[...] = n
    x3_s[...] = jnp.transpose(xc[:3])
    rows = pl.ds(i * BN, BN)
    cb = c_s[rows, :]
    nb = n_s[rows, :]
    c_out[0] = cb
    n_out[0] = nb
    idxc_out[0] = _topk_idx(_f2key(_pd_block(cb, c, i)))
    idxn_out[0] = _topk_idx(_f2key(_pd_block(nb, n, i)))
    xct = jnp.transpose(xc[:3])
    ka = _f2key(_pd_block(x3_s[rows, :], xct, i))
    ka_out[0] = ka
    lo0 = jnp.min(ka, axis=1, keepdims=True)
    hi0 = jnp.max(ka, axis=1, keepdims=True)

    def body(_, carry):
        lo, hi = carry
        mid = lo + lax.shift_right_logical(hi - lo + 1, 1)
        cnt = jnp.sum((ka >= mid).astype(jnp.int32), axis=1, keepdims=True)
        ge = cnt >= KADJ
        return jnp.where(ge, mid, lo), jnp.where(ge, hi, mid - 1)

    lo, _ = lax.fori_loop(0, 32, body, (lo0, hi0))
    tk_out[0] = jnp.reshape(lo, (1, BN))


def _edge_kernel(g_ref, ctr_ref, w1a, w1b, g1, b1, w2, g2, b2,
                 w3a, w3b, g3, b3, w4, g4, b4, w5a, w5b, g5, b5, xs_out):
    G = g_ref[0][:, :DIM]
    C = ctr_ref[0]
    CB = jnp.broadcast_to(C[:, None, :], (BN, K1, DIM)).reshape(BN * K1, DIM)
    D = G - CB
    h = _lrelu((_dot_t(D, w1a[...]) + _dot_t(CB, w1b[...])) / SQ * g1[...] + b1[...])
    h = _lrelu(_dot_t(h, w2[...]) / SQ * g2[...] + b2[...])
    xk1 = jnp.max(h.reshape(BN, K1, DIM), axis=1)
    G2 = G.reshape(BN, K1, DIM)[:, :K2, :].reshape(BN * K2, DIM)
    CB2 = jnp.broadcast_to(C[:, None, :], (BN, K2, DIM)).reshape(BN * K2, DIM)
    D2 = G2 - CB2
    h3 = _lrelu((_dot_t(D2, w3a[...]) + _dot_t(CB2, w3b[...])) / SQ * g3[...] + b3[...])
    h4 = _lrelu(_dot_t(h3, w4[...]) / SQ * g4[...] + b4[...])
    xk1r = jnp.broadcast_to(xk1[:, None, :], (BN, K2, DIM)).reshape(BN * K2, DIM)
    h5 = _lrelu((_dot_t(h4, w5a[...]) + _dot_t(xk1r, w5b[...])) / SQ * g5[...] + b5[...])
    xs_out[0] = jnp.max(h5.reshape(BN, K2, DIM), axis=1)


def _adj_kernel(k_ref, kt_ref, trow_ref, tcol_ref, a_out):
    kb = k_ref[0]
    ktb = jnp.transpose(kt_ref[0], (1, 0))
    m = (kb >= tcol_ref[0]) | (ktb >= trow_ref[0])
    a_out[0] = m.astype(jnp.float32)


def _gcn(xs, A, wh, gh, bh, lw, wt, gt, bt):
    h = _lrelu(_dot_t(xs, wh) / SQ * gh + bh)
    sc = h
    for (w1, g, b, w2) in lw:
        tt = _lrelu(_dot_t(h, w1) / SQ * g + b)
        tt = _dot_t(tt, w2)
        h = lax.dot_general(A, tt, (((0,), (0,)), ((), ())),
                            preferred_element_type=jnp.float32)
    return _lrelu(_dot_t(h, wt) / SQ * gt + bt) + sc


def _final_kernel(xsc_ref, xsn_ref, a_ref,
                  whc, ghc, bhc, c10, cg0, cb0, c20, c11, cg1, cb1, c21,
                  wtc, gtc, btc,
                  whn, ghn, bhn, n10, ng0, nb0, n20, n11, ng1, nb1, n21,
                  wtn, gtn, btn,
                  wsc, bsc, wsn, bsn, wse1, wse2, wpred, bpred, out_ref):
    A = a_ref[0]
    xc = _gcn(xsc_ref[0], A, whc[...], ghc[...], bhc[...],
              [(c10[...], cg0[...], cb0[...], c20[...]),
               (c11[...], cg1[...], cb1[...], c21[...])],
              wtc[...], gtc[...], btc[...])
    xn = _gcn(xsn_ref[0], A, whn[...], ghn[...], bhn[...],
              [(n10[...], ng0[...], nb0[...], n20[...]),
               (n11[...], ng1[...], nb1[...], n21[...])],
              wtn[...], gtn[...], btn[...])
    zc = jnp.sum(xc * wsc[...], axis=1, keepdims=True)
    zn = jnp.sum(xn * wsn[...], axis=1, keepdims=True)
    ac = _sigmoid(jnp.broadcast_to(zc, (N, FEAT)) + bsc[...])
    an = _sigmoid(jnp.broadcast_to(zn, (N, FEAT)) + bsn[...])
    xpn = xn + xn * ac
    xpc = xc + xc * an
    fs = xpc + xpn
    y = jnp.mean(fs, axis=0, keepdims=True)
    y = jnp.maximum(_dot_t(y, wse1[...]), 0.0)
    y = _sigmoid(_dot_t(y, wse2[...]))
    f = fs * y
    out_ref[0] = _dot_t(f, wpred[...]) + bpred[...]


NW = 32
GCH = 128
NIDX = B * N * K1


def _sc_gather_body(table_ref, idx_ref, out_ref, idx_v, rows_v, sem):
    wid = lax.axis_index("s") * 2 + lax.axis_index("c")
    per_w = NIDX // NW

    def body(ci, carry):
        base = wid * per_w + ci * GCH
        pltpu.sync_copy(idx_ref.at[pl.ds(base, GCH)], idx_v)
        pltpu.async_copy(table_ref.at[idx_v], rows_v, sem).wait()
        pltpu.sync_copy(rows_v, out_ref.at[pl.ds(base, GCH)])
        return carry

    lax.fori_loop(0, per_w // GCH, body, 0)


def _sc_gather(table, idx):
    k = pl.kernel(
        _sc_gather_body,
        mesh=plsc.VectorSubcoreMesh(core_axis_name="c", subcore_axis_name="s"),
        out_type=jax.ShapeDtypeStruct((NIDX, FEAT), jnp.float32),
        scratch_types=[
            pltpu.VMEM((GCH,), jnp.int32),
            pltpu.VMEM((GCH, FEAT), jnp.float32),
            pltpu.SemaphoreType.DMA,
        ],
    )
    return k(table, idx)


def _full(bs):
    return pl.BlockSpec(bs, lambda b, i: (0,) * len(bs))


def _v(p):
    return p.reshape(1, -1)


def kernel(x, params):
    hc, hn = params['head_c'], params['head_n']
    c_nm, n_nm, idx_c, idx_n, ka, tk = pl.pallas_call(
        _prep_kernel,
        grid=(B, NBLK),
        in_specs=[pl.BlockSpec((1, 2 * IN_CH, N), lambda b, i: (b, 0, 0))]
        + [_full(s) for s in [(DIM, IN_CH), (1, DIM), (1, DIM), (DIM, DIM)] * 2],
        out_specs=[
            pl.BlockSpec((1, BN, DIM), lambda b, i: (b, i, 0)),
            pl.BlockSpec((1, BN, DIM), lambda b, i: (b, i, 0)),
            pl.BlockSpec((1, BN, K1), lambda b, i: (b, i, 0)),
            pl.BlockSpec((1, BN, K1), lambda b, i: (b, i, 0)),
            pl.BlockSpec((1, BN, N), lambda b, i: (b, i, 0)),
            pl.BlockSpec((1, 1, BN), lambda b, i: (b * NBLK + i, 0, 0)),
        ],
        out_shape=[
            jax.ShapeDtypeStruct((B, N, DIM), jnp.float32),
            jax.ShapeDtypeStruct((B, N, DIM), jnp.float32),
            jax.ShapeDtypeStruct((B, N, K1), jnp.int32),
            jax.ShapeDtypeStruct((B, N, K1), jnp.int32),
            jax.ShapeDtypeStruct((B, N, N), jnp.int32),
            jax.ShapeDtypeStruct((B * NBLK, 1, BN), jnp.int32),
        ],
        scratch_shapes=[
            pltpu.VMEM((N, DIM), jnp.float32),
            pltpu.VMEM((N, DIM), jnp.float32),
            pltpu.VMEM((N, 3), jnp.float32),
        ],
    )(x, hc['w1'], _v(hc['g1']), _v(hc['b1']), hc['w2'],
      hn['w1'], _v(hn['g1']), _v(hn['b1']), hn['w2'])

    boff = (jnp.arange(B, dtype=jnp.int32) * N)[:, None, None]
    pad = jnp.zeros((B * N, FEAT - DIM), jnp.float32)
    tab_c = jnp.concatenate([c_nm.reshape(B * N, DIM), pad], axis=1)
    tab_n = jnp.concatenate([n_nm.reshape(B * N, DIM), pad], axis=1)
    gath_c = _sc_gather(tab_c, (idx_c + boff).reshape(NIDX)
                        ).reshape(B, N * K1, FEAT)
    gath_n = _sc_gather(tab_n, (idx_n + boff).reshape(NIDX)
                        ).reshape(B, N * K1, FEAT)

    def edge(gath, ctr, p):
        ws = []
        for nm, cin in zip(['w1', 'w2', 'w3', 'w4', 'w5'],
                           [2 * DIM, DIM, 2 * DIM, DIM, 2 * DIM]):
            w = p[nm]
            if cin == 2 * DIM:
                ws += [w[:, :DIM], w[:, DIM:]]
            else:
                ws += [w]
        args = [ws[0], ws[1], _v(p['g1']), _v(p['b1']), ws[2], _v(p['g2']), _v(p['b2']),
                ws[3], ws[4], _v(p['g3']), _v(p['b3']), ws[5], _v(p['g4']), _v(p['b4']),
                ws[6], ws[7], _v(p['g5']), _v(p['b5'])]
        return pl.pallas_call(
            _edge_kernel,
            grid=(B, NBLK),
            in_specs=[pl.BlockSpec((1, BN * K1, FEAT), lambda b, i: (b, i, 0)),
                      pl.BlockSpec((1, BN, DIM), lambda b, i: (b, i, 0))]
            + [_full(a.shape) for a in args],
            out_specs=pl.BlockSpec((1, BN, DIM), lambda b, i: (b, i, 0)),
            out_shape=jax.ShapeDtypeStruct((B, N, DIM), jnp.float32),
        )(gath, ctr, *args)

    xs_c = edge(gath_c, c_nm, params['mgm_c'])
    xs_n = edge(gath_n, n_nm, params['mgm_n'])

    t = tk.reshape(B, N)
    trow = t.reshape(B, 1, N)
    tcol = t.reshape(B, N, 1)
    A = pl.pallas_call(
        _adj_kernel,
        grid=(B, NBLK),
        in_specs=[pl.BlockSpec((1, BN, N), lambda b, i: (b, i, 0)),
                  pl.BlockSpec((1, N, BN), lambda b, i: (b, 0, i)),
                  pl.BlockSpec((1, 1, N), lambda b, i: (b, 0, 0)),
                  pl.BlockSpec((1, BN, 1), lambda b, i: (b, i, 0))],
        out_specs=pl.BlockSpec((1, BN, N), lambda b, i: (b, i, 0)),
        out_shape=jax.ShapeDtypeStruct((B, N, N), jnp.float32),
    )(ka, ka, trow, tcol)

    gc, gn, fu = params['gcn_c'], params['gcn_n'], params['fuse']
    gargs = []
    for g in (gc, gn):
        gargs += [g['wh'], _v(g['gh']), _v(g['bh'])]
        for l in g['layers']:
            gargs += [l['w1'], _v(l['g']), _v(l['b']), l['w2']]
        gargs += [g['wt'], _v(g['gt']), _v(g['bt'])]
    fargs = [fu['wsc'], jnp.broadcast_to(_v(fu['bsc']), (1, FEAT)),
             fu['wsn'], jnp.broadcast_to(_v(fu['bsn']), (1, FEAT)),
             fu['wse1'], fu['wse2'], params['wpred'], _v(params['bpred'])]
    allargs = gargs + fargs
    outs = []
    for b in range(B):
        outs.append(pl.pallas_call(
            _final_kernel,
            grid=(1,),
            in_specs=[pl.BlockSpec((1, N, DIM), lambda i: (0, 0, 0)),
                      pl.BlockSpec((1, N, DIM), lambda i: (0, 0, 0)),
                      pl.BlockSpec((1, N, N), lambda i: (0, 0, 0))]
            + [pl.BlockSpec(a.shape, lambda i: (0,) * a.ndim) for a in allargs],
            out_specs=pl.BlockSpec((1, N, OUT_CH), lambda i: (0, 0, 0)),
            out_shape=jax.ShapeDtypeStruct((1, N, OUT_CH), jnp.float32),
        )(xs_c[b:b + 1], xs_n[b:b + 1], A[b:b + 1], *allargs))
    out_nm = jnp.concatenate(outs, axis=0)
    return jnp.transpose(out_nm, (0, 2, 1))
